# baseline (speedup 1.0000x reference)
import functools

import jax
import jax.numpy as jnp
from jax import lax
from jax.experimental import pallas as pl
from jax.experimental.pallas import tpu as pltpu
from jax.experimental.pallas import tpu_sc as plsc

BATCH = 1024
TIME = 50
D = 1000
NC, NS = 2, 16
NW = NC * NS
B_PER_W = BATCH // NW

_mesh = plsc.VectorSubcoreMesh(core_axis_name="c", subcore_axis_name="s")


@functools.partial(
    pl.kernel,
    mesh=_mesh,
    out_type=jax.ShapeDtypeStruct((BATCH, TIME, D), jnp.float32),
    scratch_types=[
        pltpu.VMEM((B_PER_W, TIME), jnp.int32),
        pltpu.VMEM((TIME, D), jnp.float32),
        pltpu.VMEM((TIME, D), jnp.float32),
        pltpu.SemaphoreType.DMA,
        pltpu.SemaphoreType.DMA,
        pltpu.SemaphoreType.DMA,
        pltpu.SemaphoreType.DMA,
    ],
    compiler_params=pltpu.CompilerParams(use_tc_tiling_on_sc=False),
)
def _embed(idx_hbm, table_hbm, out_hbm, idx_v, buf0, buf1, g0, g1, w0, w1):
    wid = lax.axis_index("s") * NC + lax.axis_index("c")
    base = wid * B_PER_W

    pltpu.sync_copy(idx_hbm.at[wid], idx_v)

    def gather_start(c, buf, sem):
        return pltpu.async_copy(table_hbm.at[idx_v.at[c]], buf, sem)

    def gather_wait(c, buf, sem):
        pltpu.make_async_copy(table_hbm.at[idx_v.at[c]], buf, sem).wait()

    def write_start(c, buf, sem):
        return pltpu.async_copy(buf, out_hbm.at[base + c], sem)

    def write_wait(c, buf, sem):
        pltpu.make_async_copy(buf, out_hbm.at[base + c], sem).wait()

    gather_start(0, buf0, g0)
    gather_start(1, buf1, g1)

    def body(j, carry):
        c0 = 2 * j
        gather_wait(c0, buf0, g0)
        write_start(c0, buf0, w0)
        gather_wait(c0 + 1, buf1, g1)
        write_start(c0 + 1, buf1, w1)
        write_wait(c0, buf0, w0)
        gather_start(c0 + 2, buf0, g0)
        write_wait(c0 + 1, buf1, w1)
        gather_start(c0 + 3, buf1, g1)
        return carry

    lax.fori_loop(0, B_PER_W // 2 - 1, body, 0)

    cL = B_PER_W - 2
    gather_wait(cL, buf0, g0)
    hw0 = write_start(cL, buf0, w0)
    gather_wait(cL + 1, buf1, g1)
    hw1 = write_start(cL + 1, buf1, w1)
    hw0.wait()
    hw1.wait()


def kernel(idx, table):
    idx_r = idx.reshape(NW, B_PER_W, TIME).astype(jnp.int32)
    return _embed(idx_r, table)


# speedup vs baseline: 2.0231x; 2.0231x over previous
import functools

import jax
import jax.numpy as jnp
from jax import lax
from jax.experimental import pallas as pl
from jax.experimental.pallas import tpu as pltpu
from jax.experimental.pallas import tpu_sc as plsc

BATCH = 1024
TIME = 50
D = 1024
NC, NS = 2, 16
NW = NC * NS
B_PER_W = BATCH // NW

_mesh = plsc.VectorSubcoreMesh(core_axis_name="c", subcore_axis_name="s")


@functools.partial(
    pl.kernel,
    mesh=_mesh,
    out_type=jax.ShapeDtypeStruct((BATCH, TIME, D), jnp.float32),
    scratch_types=[
        pltpu.VMEM((B_PER_W, TIME), jnp.int32),
        pltpu.VMEM((TIME, D), jnp.float32),
        pltpu.VMEM((TIME, D), jnp.float32),
        pltpu.SemaphoreType.DMA,
        pltpu.SemaphoreType.DMA,
        pltpu.SemaphoreType.DMA,
        pltpu.SemaphoreType.DMA,
    ],
    compiler_params=pltpu.CompilerParams(use_tc_tiling_on_sc=True),
)
def _embed(idx_hbm, table_hbm, out_hbm, idx_v, buf0, buf1, g0, g1, w0, w1):
    wid = lax.axis_index("s") * NC + lax.axis_index("c")
    base = wid * B_PER_W

    pltpu.sync_copy(idx_hbm.at[wid], idx_v)

    def gather_start(c, buf, sem):
        return pltpu.async_copy(table_hbm.at[idx_v.at[c]], buf, sem)

    def gather_wait(c, buf, sem):
        pltpu.make_async_copy(table_hbm.at[idx_v.at[c]], buf, sem).wait()

    def write_start(c, buf, sem):
        return pltpu.async_copy(buf, out_hbm.at[base + c], sem)

    def write_wait(c, buf, sem):
        pltpu.make_async_copy(buf, out_hbm.at[base + c], sem).wait()

    gather_start(0, buf0, g0)
    gather_start(1, buf1, g1)

    def body(j, carry):
        c0 = 2 * j
        gather_wait(c0, buf0, g0)
        write_start(c0, buf0, w0)
        gather_wait(c0 + 1, buf1, g1)
        write_start(c0 + 1, buf1, w1)
        write_wait(c0, buf0, w0)
        gather_start(c0 + 2, buf0, g0)
        write_wait(c0 + 1, buf1, w1)
        gather_start(c0 + 3, buf1, g1)
        return carry

    lax.fori_loop(0, B_PER_W // 2 - 1, body, 0)

    cL = B_PER_W - 2
    gather_wait(cL, buf0, g0)
    hw0 = write_start(cL, buf0, w0)
    gather_wait(cL + 1, buf1, g1)
    hw1 = write_start(cL + 1, buf1, w1)
    hw0.wait()
    hw1.wait()


def kernel(idx, table):
    idx_r = idx.reshape(NW, B_PER_W, TIME).astype(jnp.int32)
    import jax.numpy as _jnp
    table_p = _jnp.pad(table, ((0,0),(0,24)))
    out = _embed(idx_r, table_p)
    return out[..., :1000]
